# Initial kernel scaffold; baseline (speedup 1.0000x reference)
import functools

import jax
import jax.numpy as jnp
from jax import lax
from jax.experimental import pallas as pl
from jax.experimental.pallas import tpu as pltpu
from jax.experimental.pallas import tpu_sc as plsc

N = 10000
E = 320000
DF = 128
H = 64
H2 = 32
G = 256
P = 512

N_PAD = 10240
NC = 2
NS = 16
NW = NC * NS
EW = E // NW
CK = 128
NFULL = EW // CK
TAIL = EW - NFULL * CK
RT = N_PAD // NS

_MESH = plsc.VectorSubcoreMesh(core_axis_name="c", subcore_axis_name="s")
_F32 = jnp.float32
_HIGH = jax.lax.Precision.HIGHEST


def _zero_f32_vmem(ref, n_rows, n_cols):
    z16 = jnp.zeros((16,), _F32)

    @pl.loop(0, n_rows)
    def _(i):
        for j in range(n_cols // 16):
            ref[i, pl.ds(j * 16, 16)] = z16


def _deg_body(edge, out, dbuf, dtail, ones_ck, ones_t, zrow, acc):
    c = lax.axis_index("c")
    s = lax.axis_index("s")
    wid = c * NS + s

    for j in range(CK // 16):
        ones_ck[pl.ds(j * 16, 16)] = jnp.ones((16,), _F32)
    ones_t[pl.ds(0, 16)] = jnp.ones((16,), _F32)
    _zero_f32_vmem(zrow, 8, RT // 8)

    pltpu.sync_copy(zrow, acc.at[pl.ds(s * RT, RT)])
    plsc.subcore_barrier()

    base = wid * EW

    @pl.loop(0, NFULL)
    def _(j):
        pltpu.sync_copy(edge.at[1, pl.ds(base + j * CK, CK)], dbuf)
        pltpu.sync_copy(ones_ck, acc.at[dbuf], add=True)

    pltpu.sync_copy(edge.at[1, pl.ds(base + NFULL * CK, TAIL)], dtail)
    pltpu.sync_copy(ones_t, acc.at[dtail], add=True)

    plsc.subcore_barrier()
    pltpu.sync_copy(acc.at[pl.ds(s * RT, RT)], out.at[c, pl.ds(s * RT, RT)])


_deg_call = pl.kernel(
    _deg_body,
    out_type=jax.ShapeDtypeStruct((NC, N_PAD), _F32),
    mesh=_MESH,
    scratch_types=[
        pltpu.VMEM((CK,), jnp.int32),
        pltpu.VMEM((TAIL,), jnp.int32),
        pltpu.VMEM((CK,), _F32),
        pltpu.VMEM((TAIL,), _F32),
        pltpu.VMEM((8, RT // 8), _F32),
        pltpu.VMEM_SHARED((N_PAD,), _F32),
    ],
)


def _agg_body(D, edge, table, out, sbuf, dbuf, stail, dtail, rows, rtail,
              acc, sem):
    c = lax.axis_index("c")
    s = lax.axis_index("s")
    wid = c * NS + s

    _zero_f32_vmem(rows, CK, D)

    @pl.loop(0, RT // CK)
    def _(t):
        pltpu.sync_copy(rows, acc.at[pl.ds(s * RT + t * CK, CK), :])

    plsc.subcore_barrier()

    base = wid * EW

    @pl.loop(0, NFULL)
    def _(j):
        off = base + j * CK
        pltpu.sync_copy(edge.at[0, pl.ds(off, CK)], sbuf)
        pltpu.sync_copy(edge.at[1, pl.ds(off, CK)], dbuf)
        pltpu.async_copy(table.at[sbuf], rows, sem).wait()
        pltpu.sync_copy(rows, acc.at[dbuf], add=True)

    off = base + NFULL * CK
    pltpu.sync_copy(edge.at[0, pl.ds(off, TAIL)], stail)
    pltpu.sync_copy(edge.at[1, pl.ds(off, TAIL)], dtail)
    pltpu.async_copy(table.at[stail], rtail, sem).wait()
    pltpu.sync_copy(rtail, acc.at[dtail], add=True)

    plsc.subcore_barrier()
    pltpu.sync_copy(acc.at[pl.ds(s * RT, RT), :],
                    out.at[c, pl.ds(s * RT, RT), :])


def _make_agg(D):
    return pl.kernel(
        functools.partial(_agg_body, D),
        out_type=jax.ShapeDtypeStruct((NC, N_PAD, D), _F32),
        mesh=_MESH,
        scratch_types=[
            pltpu.VMEM((CK,), jnp.int32),
            pltpu.VMEM((CK,), jnp.int32),
            pltpu.VMEM((TAIL,), jnp.int32),
            pltpu.VMEM((TAIL,), jnp.int32),
            pltpu.VMEM((CK, D), _F32),
            pltpu.VMEM((TAIL, D), _F32),
            pltpu.VMEM_SHARED((N_PAD, D), _F32),
            pltpu.SemaphoreType.DMA,
        ],
    )


_agg64 = _make_agg(H)
_agg32 = _make_agg(H2)


def _dense1_body(deg_ref, x_ref, w1_ref, dinv_ref, hh1_ref):
    deg = deg_ref[...]
    dinv = lax.rsqrt(deg[0] + deg[1] + 1.0)
    dinv_ref[...] = dinv[:, None]
    h = jnp.dot(x_ref[...], w1_ref[...], preferred_element_type=_F32,
                precision=_HIGH)
    hh1_ref[...] = h * dinv[:N, None]


def _dense2_body(p_ref, hh1_ref, dinv_ref, b1_ref, w2_ref, hh2_ref):
    dinv = dinv_ref[...][:N]
    p = p_ref[...]
    agg = (p[0, :N] + p[1, :N] + hh1_ref[...]) * dinv
    t1 = jnp.tanh(agg + b1_ref[...])
    hh2_ref[...] = jnp.dot(t1, w2_ref[...], preferred_element_type=_F32,
                           precision=_HIGH) * dinv


def _dense3_body(p_ref, hh2_ref, dinv_ref, b2_ref, wf1_ref, bf1_ref,
                 wf2_ref, bf2_ref, batch_ref, ia_ref, ib_ref,
                 out_ref, xu_ref):
    dinv = dinv_ref[...][:N]
    p = p_ref[...]
    agg = (p[0, :N] + p[1, :N] + hh2_ref[...]) * dinv
    t2 = jnp.tanh(agg + b2_ref[...])
    w = jnp.dot(wf1_ref[...], wf2_ref[...], preferred_element_type=_F32,
                precision=_HIGH)
    cc = jnp.dot(bf1_ref[...], wf2_ref[...], preferred_element_type=_F32,
                 precision=_HIGH) + bf2_ref[...]
    z = jnp.dot(t2, w, preferred_element_type=_F32, precision=_HIGH) + cc

    batch = batch_ref[...]
    acc = jnp.zeros((G, 2), _F32)
    CH = 1250
    for k in range(N // CH):
        bc = batch[:, k * CH:(k + 1) * CH]
        m = (lax.broadcasted_iota(jnp.int32, (G, CH), 0) == bc).astype(_F32)
        zc = z[k * CH:(k + 1) * CH]
        zo = jnp.concatenate([zc, jnp.ones_like(zc)], axis=1)
        acc = acc + jnp.dot(m, zo, preferred_element_type=_F32,
                            precision=_HIGH)
    u = acc[:, 0:1] / jnp.maximum(acc[:, 1:2], 1.0)
    xu_ref[...] = u

    gi = lax.broadcasted_iota(jnp.int32, (P, G), 1)
    a = (ib_ref[...] == gi).astype(_F32) - (ia_ref[...] == gi).astype(_F32)
    out_ref[...] = jnp.dot(a, u, preferred_element_type=_F32,
                           precision=_HIGH)


_dense1 = pl.pallas_call(
    _dense1_body,
    out_shape=[jax.ShapeDtypeStruct((N_PAD, 1), _F32),
               jax.ShapeDtypeStruct((N, H), _F32)],
)

_dense2 = pl.pallas_call(
    _dense2_body,
    out_shape=jax.ShapeDtypeStruct((N, H2), _F32),
)

_dense3 = pl.pallas_call(
    _dense3_body,
    out_shape=[jax.ShapeDtypeStruct((P, 1), _F32),
               jax.ShapeDtypeStruct((G, 1), _F32)],
)


def kernel(x, edge_index, batch, idx_a, idx_b, W1, b1, W2, b2,
           Wf1, bf1, Wf2, bf2):
    degp = _deg_call(edge_index)
    dinv2d, hh1 = _dense1(degp, x, W1)
    p1 = _agg64(edge_index, hh1)
    hh2 = _dense2(p1, hh1, dinv2d, b1.reshape(1, H), W2)
    p2 = _agg32(edge_index, hh2)
    out2d, xu = _dense3(p2, hh2, dinv2d, b2.reshape(1, H2), Wf1,
                        bf1.reshape(1, H2), Wf2, bf2.reshape(1, 1),
                        batch.reshape(1, N), idx_a.reshape(P, 1),
                        idx_b.reshape(P, 1))
    return out2d.reshape(P), xu


# speedup vs baseline: 19.3345x; 19.3345x over previous
import functools

import jax
import jax.numpy as jnp
from jax import lax
from jax.experimental import pallas as pl
from jax.experimental.pallas import tpu as pltpu
from jax.experimental.pallas import tpu_sc as plsc

N = 10000
E = 320000
DF = 128
H = 64
H2 = 32
G = 256
P = 512

N_PAD = 10240
NC = 2
NS = 16
NW = NC * NS
EW = E // NW
CK = 128
NFULL = EW // CK
TAIL = EW - NFULL * CK
RT = N_PAD // NS

_MESH = plsc.VectorSubcoreMesh(core_axis_name="c", subcore_axis_name="s")
_F32 = jnp.float32
_HIGH = jax.lax.Precision.HIGHEST


def _zero_f32_vmem(ref, n_rows, n_cols):
    z16 = jnp.zeros((16,), _F32)

    @pl.loop(0, n_rows)
    def _(i):
        for j in range(n_cols // 16):
            ref[i, pl.ds(j * 16, 16)] = z16


def _zero_f32_vmem_1d(ref, n):
    z16 = jnp.zeros((16,), _F32)

    @pl.loop(0, n // 16)
    def _(i):
        ref[pl.ds(i * 16, 16)] = z16


def _deg_body(dst, out, dbuf, dtail, ones_ck, ones_t, zrow, acc):
    c = lax.axis_index("c")
    s = lax.axis_index("s")
    wid = c * NS + s

    for j in range(CK // 16):
        ones_ck[pl.ds(j * 16, 16)] = jnp.ones((16,), _F32)
    ones_t[pl.ds(0, 16)] = jnp.ones((16,), _F32)
    _zero_f32_vmem_1d(zrow, RT)

    pltpu.sync_copy(zrow, acc.at[pl.ds(s * RT, RT)])
    plsc.subcore_barrier()

    base = wid * EW

    @pl.loop(0, NFULL)
    def _(j):
        pltpu.sync_copy(dst.at[pl.ds(base + j * CK, CK)], dbuf)
        pltpu.sync_copy(ones_ck, acc.at[dbuf], add=True)

    pltpu.sync_copy(dst.at[pl.ds(base + NFULL * CK, TAIL)], dtail)
    pltpu.sync_copy(ones_t, acc.at[dtail], add=True)

    plsc.subcore_barrier()
    pltpu.sync_copy(acc.at[pl.ds(s * RT, RT)], out.at[c, pl.ds(s * RT, RT)])


_SC_PARAMS = pltpu.CompilerParams(use_tc_tiling_on_sc=False)

_deg_call = pl.kernel(
    _deg_body,
    out_type=jax.ShapeDtypeStruct((NC, N_PAD), _F32),
    mesh=_MESH,
    compiler_params=_SC_PARAMS,
    scratch_types=[
        pltpu.VMEM((CK,), jnp.int32),
        pltpu.VMEM((TAIL,), jnp.int32),
        pltpu.VMEM((CK,), _F32),
        pltpu.VMEM((TAIL,), _F32),
        pltpu.VMEM((RT,), _F32),
        pltpu.VMEM_SHARED((N_PAD,), _F32),
    ],
)


def _agg_body(D, src, dst, table, out, sbuf, dbuf, stail, dtail, rows, rtail,
              acc, sem):
    c = lax.axis_index("c")
    s = lax.axis_index("s")
    wid = c * NS + s

    _zero_f32_vmem(rows, CK, D)

    @pl.loop(0, RT // CK)
    def _(t):
        pltpu.sync_copy(rows, acc.at[pl.ds(s * RT + t * CK, CK), :])

    plsc.subcore_barrier()

    base = wid * EW

    @pl.loop(0, NFULL)
    def _(j):
        off = base + j * CK
        pltpu.sync_copy(src.at[pl.ds(off, CK)], sbuf)
        pltpu.sync_copy(dst.at[pl.ds(off, CK)], dbuf)
        pltpu.async_copy(table.at[sbuf], rows, sem).wait()
        pltpu.sync_copy(rows, acc.at[dbuf], add=True)

    off = base + NFULL * CK
    pltpu.sync_copy(src.at[pl.ds(off, TAIL)], stail)
    pltpu.sync_copy(dst.at[pl.ds(off, TAIL)], dtail)
    pltpu.async_copy(table.at[stail], rtail, sem).wait()
    pltpu.sync_copy(rtail, acc.at[dtail], add=True)

    plsc.subcore_barrier()
    pltpu.sync_copy(acc.at[pl.ds(s * RT, RT), :],
                    out.at[c, pl.ds(s * RT, RT), :])


def _make_agg(D):
    return pl.kernel(
        functools.partial(_agg_body, D),
        out_type=jax.ShapeDtypeStruct((NC, N_PAD, D), _F32),
        mesh=_MESH,
        compiler_params=_SC_PARAMS,
        scratch_types=[
            pltpu.VMEM((CK,), jnp.int32),
            pltpu.VMEM((CK,), jnp.int32),
            pltpu.VMEM((TAIL,), jnp.int32),
            pltpu.VMEM((TAIL,), jnp.int32),
            pltpu.VMEM((CK, D), _F32),
            pltpu.VMEM((TAIL, D), _F32),
            pltpu.VMEM_SHARED((N_PAD, D), _F32),
            pltpu.SemaphoreType.DMA,
        ],
    )


_agg64 = _make_agg(H)
_agg32 = _make_agg(H2)


def _dense1_body(deg_ref, x_ref, w1_ref, dinv_ref, hh1_ref):
    deg = deg_ref[...]
    dinv = lax.rsqrt(deg[0] + deg[1] + 1.0)
    dinv_ref[...] = dinv[:, None]
    h = jnp.dot(x_ref[...], w1_ref[...], preferred_element_type=_F32,
                precision=_HIGH)
    hh1_ref[...] = h * dinv[:N, None]


def _dense2_body(p_ref, hh1_ref, dinv_ref, b1_ref, w2_ref, hh2_ref):
    dinv = dinv_ref[...][:N]
    p = p_ref[...]
    agg = (p[0, :N] + p[1, :N] + hh1_ref[...]) * dinv
    t1 = jnp.tanh(agg + b1_ref[...])
    hh2_ref[...] = jnp.dot(t1, w2_ref[...], preferred_element_type=_F32,
                           precision=_HIGH) * dinv


def _dense3_body(p_ref, hh2_ref, dinv_ref, b2_ref, wf1_ref, bf1_ref,
                 wf2_ref, bf2_ref, batch_ref, ia_ref, ib_ref,
                 out_ref, xu_ref):
    dinv = dinv_ref[...][:N]
    p = p_ref[...]
    agg = (p[0, :N] + p[1, :N] + hh2_ref[...]) * dinv
    t2 = jnp.tanh(agg + b2_ref[...])
    w = jnp.dot(wf1_ref[...], wf2_ref[...], preferred_element_type=_F32,
                precision=_HIGH)
    cc = jnp.dot(bf1_ref[...], wf2_ref[...], preferred_element_type=_F32,
                 precision=_HIGH) + bf2_ref[...]
    z = jnp.dot(t2, w, preferred_element_type=_F32, precision=_HIGH) + cc

    batch = batch_ref[...]
    acc = jnp.zeros((G, 2), _F32)
    CH = 1250
    for k in range(N // CH):
        bc = batch[:, k * CH:(k + 1) * CH]
        m = (lax.broadcasted_iota(jnp.int32, (G, CH), 0) == bc).astype(_F32)
        zc = z[k * CH:(k + 1) * CH]
        zo = jnp.concatenate([zc, jnp.ones_like(zc)], axis=1)
        acc = acc + jnp.dot(m, zo, preferred_element_type=_F32,
                            precision=_HIGH)
    u = acc[:, 0:1] / jnp.maximum(acc[:, 1:2], 1.0)
    xu_ref[...] = u

    gi = lax.broadcasted_iota(jnp.int32, (P, G), 1)
    a = (ib_ref[...] == gi).astype(_F32) - (ia_ref[...] == gi).astype(_F32)
    out_ref[...] = jnp.dot(a, u, preferred_element_type=_F32,
                           precision=_HIGH)


_dense1 = pl.pallas_call(
    _dense1_body,
    out_shape=[jax.ShapeDtypeStruct((N_PAD, 1), _F32),
               jax.ShapeDtypeStruct((N, H), _F32)],
)

_dense2 = pl.pallas_call(
    _dense2_body,
    out_shape=jax.ShapeDtypeStruct((N, H2), _F32),
)

_dense3 = pl.pallas_call(
    _dense3_body,
    out_shape=[jax.ShapeDtypeStruct((P, 1), _F32),
               jax.ShapeDtypeStruct((G, 1), _F32)],
)


def kernel(x, edge_index, batch, idx_a, idx_b, W1, b1, W2, b2,
           Wf1, bf1, Wf2, bf2):
    src = edge_index[0]
    dst = edge_index[1]
    degp = _deg_call(dst)
    dinv2d, hh1 = _dense1(degp, x, W1)
    p1 = _agg64(src, dst, hh1)
    hh2 = _dense2(p1, hh1, dinv2d, b1.reshape(1, H), W2)
    p2 = _agg32(src, dst, hh2)
    out2d, xu = _dense3(p2, hh2, dinv2d, b2.reshape(1, H2), Wf1,
                        bf1.reshape(1, H2), Wf2, bf2.reshape(1, 1),
                        batch.reshape(1, N), idx_a.reshape(P, 1),
                        idx_b.reshape(P, 1))
    return out2d.reshape(P), xu


# speedup vs baseline: 37.3424x; 1.9314x over previous
import functools

import jax
import jax.numpy as jnp
from jax import lax
from jax.experimental import pallas as pl
from jax.experimental.pallas import tpu as pltpu
from jax.experimental.pallas import tpu_sc as plsc

N = 10000
E = 320000
DF = 128
H = 64
H2 = 32
G = 256
P = 512

N_PAD = 10240
NC = 2
NS = 16
NW = NC * NS
CK = 128
NCH = E // CK
NFULL = NCH // NW
NEXTRA = NCH - NFULL * NW
SB = 13
NSUP = NFULL // SB
RT = N_PAD // NS

_MESH = plsc.VectorSubcoreMesh(core_axis_name="c", subcore_axis_name="s")
_F32 = jnp.float32
_HIGH = jax.lax.Precision.HIGHEST


def _zero_f32_vmem(ref, n_rows, n_cols):
    z16 = jnp.zeros((16,), _F32)

    @pl.loop(0, n_rows)
    def _(i):
        for j in range(n_cols // 16):
            ref[i, pl.ds(j * 16, 16)] = z16


def _zero_f32_vmem_1d(ref, n):
    z16 = jnp.zeros((16,), _F32)

    @pl.loop(0, n // 16)
    def _(i):
        ref[pl.ds(i * 16, 16)] = z16


def _deg_body(dst2, out, dbuf, ones_ck, zrow, acc):
    c = lax.axis_index("c")
    s = lax.axis_index("s")
    wid = c * NS + s

    for j in range(CK // 16):
        ones_ck[pl.ds(j * 16, 16)] = jnp.ones((16,), _F32)
    _zero_f32_vmem_1d(zrow, RT)

    pltpu.sync_copy(zrow, acc.at[pl.ds(s * RT, RT)])
    plsc.subcore_barrier()

    base_row = wid * NFULL

    @pl.loop(0, NSUP)
    def _(si):
        pltpu.sync_copy(dst2.at[pl.ds(base_row + si * SB, SB), :], dbuf)
        for b in range(SB):
            pltpu.sync_copy(ones_ck, acc.at[dbuf.at[b]], add=True)

    @pl.when(wid < NEXTRA)
    def _():
        pltpu.sync_copy(dst2.at[NFULL * NW + wid], dbuf.at[0])
        pltpu.sync_copy(ones_ck, acc.at[dbuf.at[0]], add=True)

    plsc.subcore_barrier()
    pltpu.sync_copy(acc.at[pl.ds(s * RT, RT)], out.at[c, pl.ds(s * RT, RT)])


_SC_PARAMS = pltpu.CompilerParams(use_tc_tiling_on_sc=False)

_deg_call = pl.kernel(
    _deg_body,
    out_type=jax.ShapeDtypeStruct((NC, N_PAD), _F32),
    mesh=_MESH,
    compiler_params=_SC_PARAMS,
    scratch_types=[
        pltpu.VMEM((SB, CK), jnp.int32),
        pltpu.VMEM((CK,), _F32),
        pltpu.VMEM((RT,), _F32),
        pltpu.VMEM_SHARED((N_PAD,), _F32),
    ],
)


def _agg_body(D, src2, dst2, table, out, sbuf, dbuf, rows0, rows1,
              acc, sem0, sem1):
    c = lax.axis_index("c")
    s = lax.axis_index("s")
    wid = c * NS + s

    _zero_f32_vmem(rows0, CK, D)

    @pl.loop(0, RT // CK)
    def _(t):
        pltpu.sync_copy(rows0, acc.at[pl.ds(s * RT + t * CK, CK), :])

    plsc.subcore_barrier()

    base_row = wid * NFULL
    rbufs = (rows0, rows1)
    sems = (sem0, sem1)

    @pl.loop(0, NSUP)
    def _(si):
        pltpu.sync_copy(src2.at[pl.ds(base_row + si * SB, SB), :], sbuf)
        pltpu.sync_copy(dst2.at[pl.ds(base_row + si * SB, SB), :], dbuf)
        pltpu.async_copy(table.at[sbuf.at[0]], rbufs[0], sems[0])
        for b in range(SB):
            if b + 1 < SB:
                pltpu.async_copy(table.at[sbuf.at[b + 1]],
                                 rbufs[(b + 1) % 2], sems[(b + 1) % 2])
            pltpu.make_async_copy(table.at[sbuf.at[b]], rbufs[b % 2],
                                  sems[b % 2]).wait()
            pltpu.sync_copy(rbufs[b % 2], acc.at[dbuf.at[b]], add=True)

    @pl.when(wid < NEXTRA)
    def _():
        pltpu.sync_copy(src2.at[NFULL * NW + wid], sbuf.at[0])
        pltpu.sync_copy(dst2.at[NFULL * NW + wid], dbuf.at[0])
        pltpu.async_copy(table.at[sbuf.at[0]], rows0, sem0).wait()
        pltpu.sync_copy(rows0, acc.at[dbuf.at[0]], add=True)

    plsc.subcore_barrier()
    pltpu.sync_copy(acc.at[pl.ds(s * RT, RT), :],
                    out.at[c, pl.ds(s * RT, RT), :])


def _make_agg(D):
    return pl.kernel(
        functools.partial(_agg_body, D),
        out_type=jax.ShapeDtypeStruct((NC, N_PAD, D), _F32),
        mesh=_MESH,
        compiler_params=_SC_PARAMS,
        scratch_types=[
            pltpu.VMEM((SB, CK), jnp.int32),
            pltpu.VMEM((SB, CK), jnp.int32),
            pltpu.VMEM((CK, D), _F32),
            pltpu.VMEM((CK, D), _F32),
            pltpu.VMEM_SHARED((N_PAD, D), _F32),
            pltpu.SemaphoreType.DMA,
            pltpu.SemaphoreType.DMA,
        ],
    )


_agg64 = _make_agg(H)
_agg32 = _make_agg(H2)


def _dense1_body(deg_ref, x_ref, w1_ref, dinv_ref, hh1_ref):
    deg = deg_ref[...]
    dinv = lax.rsqrt(deg[0] + deg[1] + 1.0)
    dinv_ref[...] = dinv[:, None]
    h = jnp.dot(x_ref[...], w1_ref[...], preferred_element_type=_F32,
                precision=_HIGH)
    hh1_ref[...] = h * dinv[:N, None]


def _dense2_body(p_ref, hh1_ref, dinv_ref, b1_ref, w2_ref, hh2_ref):
    dinv = dinv_ref[...][:N]
    p = p_ref[...]
    agg = (p[0, :N] + p[1, :N] + hh1_ref[...]) * dinv
    t1 = jnp.tanh(agg + b1_ref[...])
    hh2_ref[...] = jnp.dot(t1, w2_ref[...], preferred_element_type=_F32,
                           precision=_HIGH) * dinv


def _dense3_body(p_ref, hh2_ref, dinv_ref, b2_ref, wf1_ref, bf1_ref,
                 wf2_ref, bf2_ref, batch_ref, ia_ref, ib_ref,
                 out_ref, xu_ref):
    dinv = dinv_ref[...][:N]
    p = p_ref[...]
    agg = (p[0, :N] + p[1, :N] + hh2_ref[...]) * dinv
    t2 = jnp.tanh(agg + b2_ref[...])
    w = jnp.dot(wf1_ref[...], wf2_ref[...], preferred_element_type=_F32,
                precision=_HIGH)
    cc = jnp.dot(bf1_ref[...], wf2_ref[...], preferred_element_type=_F32,
                 precision=_HIGH) + bf2_ref[...]
    z = jnp.dot(t2, w, preferred_element_type=_F32, precision=_HIGH) + cc

    batch = batch_ref[...]
    acc = jnp.zeros((G, 2), _F32)
    CH = 1250
    for k in range(N // CH):
        bc = batch[:, k * CH:(k + 1) * CH]
        m = (lax.broadcasted_iota(jnp.int32, (G, CH), 0) == bc).astype(_F32)
        zc = z[k * CH:(k + 1) * CH]
        zo = jnp.concatenate([zc, jnp.ones_like(zc)], axis=1)
        acc = acc + jnp.dot(m, zo, preferred_element_type=_F32,
                            precision=_HIGH)
    u = acc[:, 0:1] / jnp.maximum(acc[:, 1:2], 1.0)
    xu_ref[...] = u

    gi = lax.broadcasted_iota(jnp.int32, (P, G), 1)
    a = (ib_ref[...] == gi).astype(_F32) - (ia_ref[...] == gi).astype(_F32)
    out_ref[...] = jnp.dot(a, u, preferred_element_type=_F32,
                           precision=_HIGH)


_dense1 = pl.pallas_call(
    _dense1_body,
    out_shape=[jax.ShapeDtypeStruct((N_PAD, 1), _F32),
               jax.ShapeDtypeStruct((N, H), _F32)],
)

_dense2 = pl.pallas_call(
    _dense2_body,
    out_shape=jax.ShapeDtypeStruct((N, H2), _F32),
)

_dense3 = pl.pallas_call(
    _dense3_body,
    out_shape=[jax.ShapeDtypeStruct((P, 1), _F32),
               jax.ShapeDtypeStruct((G, 1), _F32)],
)


def kernel(x, edge_index, batch, idx_a, idx_b, W1, b1, W2, b2,
           Wf1, bf1, Wf2, bf2):
    e2 = edge_index.reshape(2, NCH, CK)
    src2 = e2[0]
    dst2 = e2[1]
    degp = _deg_call(dst2)
    dinv2d, hh1 = _dense1(degp, x, W1)
    p1 = _agg64(src2, dst2, hh1)
    hh2 = _dense2(p1, hh1, dinv2d, b1.reshape(1, H), W2)
    p2 = _agg32(src2, dst2, hh2)
    out2d, xu = _dense3(p2, hh2, dinv2d, b2.reshape(1, H2), Wf1,
                        bf1.reshape(1, H2), Wf2, bf2.reshape(1, 1),
                        batch.reshape(1, N), idx_a.reshape(P, 1),
                        idx_b.reshape(P, 1))
    return out2d.reshape(P), xu


# speedup vs baseline: 39.3606x; 1.0540x over previous
import functools

import jax
import jax.numpy as jnp
from jax import lax
from jax.experimental import pallas as pl
from jax.experimental.pallas import tpu as pltpu
from jax.experimental.pallas import tpu_sc as plsc

N = 10000
E = 320000
DF = 128
H = 64
H2 = 32
G = 256
P = 512

N_PAD = 10240
NC = 2
NS = 16
NW = NC * NS
CK = 128
NCH = E // CK
NFULL = NCH // NW
NEXTRA = NCH - NFULL * NW
SB = 13
NSUP = NFULL // SB
RT = N_PAD // NS

_MESH = plsc.VectorSubcoreMesh(core_axis_name="c", subcore_axis_name="s")
_F32 = jnp.float32
_HIGH = jax.lax.Precision.HIGHEST


def _zero_f32_vmem(ref, n_rows, n_cols):
    z16 = jnp.zeros((16,), _F32)

    @pl.loop(0, n_rows)
    def _(i):
        for j in range(n_cols // 16):
            ref[i, pl.ds(j * 16, 16)] = z16


def _zero_f32_vmem_1d(ref, n):
    z16 = jnp.zeros((16,), _F32)

    @pl.loop(0, n // 16)
    def _(i):
        ref[pl.ds(i * 16, 16)] = z16


def _deg_body(dst2, out, dA, dB, ones_ck, zrow, acc, semiA, semiB, sems):
    c = lax.axis_index("c")
    s = lax.axis_index("s")
    wid = c * NS + s
    base_row = wid * NFULL

    pltpu.async_copy(dst2.at[pl.ds(base_row, SB), :], dA, semiA)
    pltpu.async_copy(dst2.at[pl.ds(base_row + SB, SB), :], dB, semiB)

    for j in range(CK // 16):
        ones_ck[pl.ds(j * 16, 16)] = jnp.ones((16,), _F32)
    _zero_f32_vmem_1d(zrow, RT)

    pltpu.sync_copy(zrow, acc.at[pl.ds(s * RT, RT)])
    plsc.subcore_barrier()

    def process(dbuf):
        descs = [pltpu.async_copy(ones_ck, acc.at[dbuf.at[b]], sems,
                                  add=True)
                 for b in range(SB)]
        for d in descs:
            d.wait()

    @pl.loop(0, NSUP // 2)
    def _(j):
        r0 = base_row + 2 * j * SB
        pltpu.make_async_copy(dst2.at[pl.ds(r0, SB), :], dA, semiA).wait()
        process(dA)

        @pl.when(2 * j + 2 < NSUP)
        def _():
            pltpu.async_copy(dst2.at[pl.ds(r0 + 2 * SB, SB), :], dA, semiA)

        pltpu.make_async_copy(dst2.at[pl.ds(r0 + SB, SB), :], dB,
                              semiB).wait()
        process(dB)

        @pl.when(2 * j + 3 < NSUP)
        def _():
            pltpu.async_copy(dst2.at[pl.ds(r0 + 3 * SB, SB), :], dB, semiB)

    @pl.when(wid < NEXTRA)
    def _():
        pltpu.sync_copy(dst2.at[NFULL * NW + wid], dA.at[0])
        pltpu.sync_copy(ones_ck, acc.at[dA.at[0]], add=True)

    plsc.subcore_barrier()
    pltpu.sync_copy(acc.at[pl.ds(s * RT, RT)], out.at[c, pl.ds(s * RT, RT)])


_SC_PARAMS = pltpu.CompilerParams(use_tc_tiling_on_sc=False)

_deg_call = pl.kernel(
    _deg_body,
    out_type=jax.ShapeDtypeStruct((NC, N_PAD), _F32),
    mesh=_MESH,
    compiler_params=_SC_PARAMS,
    scratch_types=[
        pltpu.VMEM((SB, CK), jnp.int32),
        pltpu.VMEM((SB, CK), jnp.int32),
        pltpu.VMEM((CK,), _F32),
        pltpu.VMEM((RT,), _F32),
        pltpu.VMEM_SHARED((N_PAD,), _F32),
        pltpu.SemaphoreType.DMA,
        pltpu.SemaphoreType.DMA,
        pltpu.SemaphoreType.DMA,
    ],
)


def _agg_body(D, src2, dst2, table, out, sA, dA, sB, dB, rows0, rows1,
              acc, semiA, semiB, semg0, semg1, sems0, sems1):
    c = lax.axis_index("c")
    s = lax.axis_index("s")
    wid = c * NS + s
    base_row = wid * NFULL

    pltpu.async_copy(src2.at[pl.ds(base_row, SB), :], sA, semiA)
    pltpu.async_copy(dst2.at[pl.ds(base_row, SB), :], dA, semiA)
    pltpu.async_copy(src2.at[pl.ds(base_row + SB, SB), :], sB, semiB)
    pltpu.async_copy(dst2.at[pl.ds(base_row + SB, SB), :], dB, semiB)

    _zero_f32_vmem(rows0, CK, D)

    @pl.loop(0, RT // CK)
    def _(t):
        pltpu.sync_copy(rows0, acc.at[pl.ds(s * RT + t * CK, CK), :])

    plsc.subcore_barrier()

    rbufs = (rows0, rows1)
    semg = (semg0, semg1)
    sems = (sems0, sems1)

    def process(sbuf, dbuf):
        gd = [None, None]
        sd = [None, None]
        gd[0] = pltpu.async_copy(table.at[sbuf.at[0]], rbufs[0], semg[0])
        for b in range(SB):
            p = b % 2
            q = (b + 1) % 2
            if b + 1 < SB:
                if sd[q] is not None:
                    sd[q].wait()
                gd[q] = pltpu.async_copy(table.at[sbuf.at[b + 1]],
                                         rbufs[q], semg[q])
            gd[p].wait()
            sd[p] = pltpu.async_copy(rbufs[p], acc.at[dbuf.at[b]],
                                     sems[p], add=True)
        sd[(SB - 1) % 2].wait()
        sd[SB % 2].wait()

    @pl.loop(0, NSUP // 2)
    def _(j):
        r0 = base_row + 2 * j * SB
        pltpu.make_async_copy(src2.at[pl.ds(r0, SB), :], sA, semiA).wait()
        pltpu.make_async_copy(dst2.at[pl.ds(r0, SB), :], dA, semiA).wait()
        process(sA, dA)

        @pl.when(2 * j + 2 < NSUP)
        def _():
            pltpu.async_copy(src2.at[pl.ds(r0 + 2 * SB, SB), :], sA, semiA)
            pltpu.async_copy(dst2.at[pl.ds(r0 + 2 * SB, SB), :], dA, semiA)

        pltpu.make_async_copy(src2.at[pl.ds(r0 + SB, SB), :], sB,
                              semiB).wait()
        pltpu.make_async_copy(dst2.at[pl.ds(r0 + SB, SB), :], dB,
                              semiB).wait()
        process(sB, dB)

        @pl.when(2 * j + 3 < NSUP)
        def _():
            pltpu.async_copy(src2.at[pl.ds(r0 + 3 * SB, SB), :], sB, semiB)
            pltpu.async_copy(dst2.at[pl.ds(r0 + 3 * SB, SB), :], dB, semiB)

    @pl.when(wid < NEXTRA)
    def _():
        pltpu.sync_copy(src2.at[NFULL * NW + wid], sA.at[0])
        pltpu.sync_copy(dst2.at[NFULL * NW + wid], dA.at[0])
        pltpu.async_copy(table.at[sA.at[0]], rows0, semg0).wait()
        pltpu.sync_copy(rows0, acc.at[dA.at[0]], add=True)

    plsc.subcore_barrier()
    pltpu.sync_copy(acc.at[pl.ds(s * RT, RT), :],
                    out.at[c, pl.ds(s * RT, RT), :])


def _make_agg(D):
    return pl.kernel(
        functools.partial(_agg_body, D),
        out_type=jax.ShapeDtypeStruct((NC, N_PAD, D), _F32),
        mesh=_MESH,
        compiler_params=_SC_PARAMS,
        scratch_types=[
            pltpu.VMEM((SB, CK), jnp.int32),
            pltpu.VMEM((SB, CK), jnp.int32),
            pltpu.VMEM((SB, CK), jnp.int32),
            pltpu.VMEM((SB, CK), jnp.int32),
            pltpu.VMEM((CK, D), _F32),
            pltpu.VMEM((CK, D), _F32),
            pltpu.VMEM_SHARED((N_PAD, D), _F32),
            pltpu.SemaphoreType.DMA,
            pltpu.SemaphoreType.DMA,
            pltpu.SemaphoreType.DMA,
            pltpu.SemaphoreType.DMA,
            pltpu.SemaphoreType.DMA,
            pltpu.SemaphoreType.DMA,
        ],
    )


_agg64 = _make_agg(H)
_agg32 = _make_agg(H2)


def _dense1_body(deg_ref, x_ref, w1_ref, dinv_ref, hh1_ref):
    deg = deg_ref[...]
    dinv = lax.rsqrt(deg[0] + deg[1] + 1.0)
    dinv_ref[...] = dinv[:, None]
    h = jnp.dot(x_ref[...], w1_ref[...], preferred_element_type=_F32,
                precision=_HIGH)
    hh1_ref[...] = h * dinv[:N, None]


def _dense2_body(p_ref, hh1_ref, dinv_ref, b1_ref, w2_ref, hh2_ref):
    dinv = dinv_ref[...][:N]
    p = p_ref[...]
    agg = (p[0, :N] + p[1, :N] + hh1_ref[...]) * dinv
    t1 = jnp.tanh(agg + b1_ref[...])
    hh2_ref[...] = jnp.dot(t1, w2_ref[...], preferred_element_type=_F32,
                           precision=_HIGH) * dinv


def _dense3_body(p_ref, hh2_ref, dinv_ref, b2_ref, wf1_ref, bf1_ref,
                 wf2_ref, bf2_ref, batch_ref, ia_ref, ib_ref,
                 out_ref, xu_ref):
    dinv = dinv_ref[...][:N]
    p = p_ref[...]
    agg = (p[0, :N] + p[1, :N] + hh2_ref[...]) * dinv
    t2 = jnp.tanh(agg + b2_ref[...])
    w = jnp.dot(wf1_ref[...], wf2_ref[...], preferred_element_type=_F32,
                precision=_HIGH)
    cc = jnp.dot(bf1_ref[...], wf2_ref[...], preferred_element_type=_F32,
                 precision=_HIGH) + bf2_ref[...]
    z = jnp.dot(t2, w, preferred_element_type=_F32, precision=_HIGH) + cc

    batch = batch_ref[...]
    acc = jnp.zeros((G, 2), _F32)
    CH = 1250
    for k in range(N // CH):
        bc = batch[:, k * CH:(k + 1) * CH]
        m = (lax.broadcasted_iota(jnp.int32, (G, CH), 0) == bc).astype(_F32)
        zc = z[k * CH:(k + 1) * CH]
        zo = jnp.concatenate([zc, jnp.ones_like(zc)], axis=1)
        acc = acc + jnp.dot(m, zo, preferred_element_type=_F32,
                            precision=_HIGH)
    u = acc[:, 0:1] / jnp.maximum(acc[:, 1:2], 1.0)
    xu_ref[...] = u

    gi = lax.broadcasted_iota(jnp.int32, (P, G), 1)
    a = (ib_ref[...] == gi).astype(_F32) - (ia_ref[...] == gi).astype(_F32)
    out_ref[...] = jnp.dot(a, u, preferred_element_type=_F32,
                           precision=_HIGH)


_dense1 = pl.pallas_call(
    _dense1_body,
    out_shape=[jax.ShapeDtypeStruct((N_PAD, 1), _F32),
               jax.ShapeDtypeStruct((N, H), _F32)],
)

_dense2 = pl.pallas_call(
    _dense2_body,
    out_shape=jax.ShapeDtypeStruct((N, H2), _F32),
)

_dense3 = pl.pallas_call(
    _dense3_body,
    out_shape=[jax.ShapeDtypeStruct((P, 1), _F32),
               jax.ShapeDtypeStruct((G, 1), _F32)],
)


def kernel(x, edge_index, batch, idx_a, idx_b, W1, b1, W2, b2,
           Wf1, bf1, Wf2, bf2):
    e2 = edge_index.reshape(2, NCH, CK)
    src2 = e2[0]
    dst2 = e2[1]
    degp = _deg_call(dst2)
    dinv2d, hh1 = _dense1(degp, x, W1)
    p1 = _agg64(src2, dst2, hh1)
    hh2 = _dense2(p1, hh1, dinv2d, b1.reshape(1, H), W2)
    p2 = _agg32(src2, dst2, hh2)
    out2d, xu = _dense3(p2, hh2, dinv2d, b2.reshape(1, H2), Wf1,
                        bf1.reshape(1, H2), Wf2, bf2.reshape(1, 1),
                        batch.reshape(1, N), idx_a.reshape(P, 1),
                        idx_b.reshape(P, 1))
    return out2d.reshape(P), xu


# speedup vs baseline: 42.7953x; 1.0873x over previous
import functools

import jax
import jax.numpy as jnp
from jax import lax
from jax.experimental import pallas as pl
from jax.experimental.pallas import tpu as pltpu
from jax.experimental.pallas import tpu_sc as plsc

N = 10000
E = 320000
DF = 128
H = 64
H2 = 32
G = 256
P = 512

N_PAD = 10240
NC = 2
NS = 16
NW = NC * NS
CK = 128
NCH = E // CK
NFULL = NCH // NW
NEXTRA = NCH - NFULL * NW
SB = 13
NSUP = NFULL // SB
RT = N_PAD // NS

_MESH = plsc.VectorSubcoreMesh(core_axis_name="c", subcore_axis_name="s")
_F32 = jnp.float32
_HIGH = jax.lax.Precision.HIGHEST


def _zero_f32_vmem(ref, n_rows, n_cols):
    z16 = jnp.zeros((16,), _F32)

    @pl.loop(0, n_rows)
    def _(i):
        for j in range(n_cols // 16):
            ref[i, pl.ds(j * 16, 16)] = z16


def _zero_f32_vmem_1d(ref, n):
    z16 = jnp.zeros((16,), _F32)

    @pl.loop(0, n // 16)
    def _(i):
        ref[pl.ds(i * 16, 16)] = z16


def _deg_body(dst2, out, dA, dB, ones_ck, zrow, acc, semiA, semiB, sems):
    c = lax.axis_index("c")
    s = lax.axis_index("s")
    wid = c * NS + s
    base_row = wid * NFULL

    pltpu.async_copy(dst2.at[pl.ds(base_row, SB), :], dA, semiA)
    pltpu.async_copy(dst2.at[pl.ds(base_row + SB, SB), :], dB, semiB)

    for j in range(CK // 16):
        ones_ck[pl.ds(j * 16, 16)] = jnp.ones((16,), _F32)
    _zero_f32_vmem_1d(zrow, RT)

    pltpu.sync_copy(zrow, acc.at[pl.ds(s * RT, RT)])
    plsc.subcore_barrier()

    def process(dbuf):
        descs = [pltpu.async_copy(ones_ck, acc.at[dbuf.at[b]], sems,
                                  add=True)
                 for b in range(SB)]
        for d in descs:
            d.wait()

    @pl.loop(0, NSUP // 2)
    def _(j):
        r0 = base_row + 2 * j * SB
        pltpu.make_async_copy(dst2.at[pl.ds(r0, SB), :], dA, semiA).wait()
        process(dA)

        @pl.when(2 * j + 2 < NSUP)
        def _():
            pltpu.async_copy(dst2.at[pl.ds(r0 + 2 * SB, SB), :], dA, semiA)

        pltpu.make_async_copy(dst2.at[pl.ds(r0 + SB, SB), :], dB,
                              semiB).wait()
        process(dB)

        @pl.when(2 * j + 3 < NSUP)
        def _():
            pltpu.async_copy(dst2.at[pl.ds(r0 + 3 * SB, SB), :], dB, semiB)

    @pl.when(wid < NEXTRA)
    def _():
        pltpu.sync_copy(dst2.at[NFULL * NW + wid], dA.at[0])
        pltpu.sync_copy(ones_ck, acc.at[dA.at[0]], add=True)

    plsc.subcore_barrier()
    pltpu.sync_copy(acc.at[pl.ds(s * RT, RT)], out.at[c, pl.ds(s * RT, RT)])


_SC_PARAMS = pltpu.CompilerParams(use_tc_tiling_on_sc=False)

_deg_call = pl.kernel(
    _deg_body,
    out_type=jax.ShapeDtypeStruct((NC, N_PAD), _F32),
    mesh=_MESH,
    compiler_params=_SC_PARAMS,
    scratch_types=[
        pltpu.VMEM((SB, CK), jnp.int32),
        pltpu.VMEM((SB, CK), jnp.int32),
        pltpu.VMEM((CK,), _F32),
        pltpu.VMEM((RT,), _F32),
        pltpu.VMEM_SHARED((N_PAD,), _F32),
        pltpu.SemaphoreType.DMA,
        pltpu.SemaphoreType.DMA,
        pltpu.SemaphoreType.DMA,
    ],
)


def _agg_body(D, src2, dst2, table, out, sA, dA, sB, dB, rows0, rows1,
              acc, semiA, semiB, semg0, semg1, sems0, sems1):
    c = lax.axis_index("c")
    s = lax.axis_index("s")
    wid = c * NS + s
    base_row = wid * NFULL

    pltpu.async_copy(src2.at[pl.ds(base_row, SB), :], sA, semiA)
    pltpu.async_copy(dst2.at[pl.ds(base_row, SB), :], dA, semiA)
    pltpu.async_copy(src2.at[pl.ds(base_row + SB, SB), :], sB, semiB)
    pltpu.async_copy(dst2.at[pl.ds(base_row + SB, SB), :], dB, semiB)

    _zero_f32_vmem(rows0, CK, D)

    @pl.loop(0, RT // CK)
    def _(t):
        pltpu.sync_copy(rows0, acc.at[pl.ds(s * RT + t * CK, CK), :])

    plsc.subcore_barrier()

    rbufs = (rows0, rows1)
    semg = (semg0, semg1)
    sems = (sems0, sems1)

    def process(sbuf, dbuf):
        gd = [None, None]
        sd = [None, None]
        gd[0] = pltpu.async_copy(table.at[sbuf.at[0]], rbufs[0], semg[0])
        for b in range(SB):
            p = b % 2
            q = (b + 1) % 2
            if b + 1 < SB:
                if sd[q] is not None:
                    sd[q].wait()
                gd[q] = pltpu.async_copy(table.at[sbuf.at[b + 1]],
                                         rbufs[q], semg[q])
            gd[p].wait()
            sd[p] = pltpu.async_copy(rbufs[p], acc.at[dbuf.at[b]],
                                     sems[p], add=True)
        sd[(SB - 1) % 2].wait()
        sd[SB % 2].wait()

    @pl.loop(0, NSUP // 2)
    def _(j):
        r0 = base_row + 2 * j * SB
        pltpu.make_async_copy(src2.at[pl.ds(r0, SB), :], sA, semiA).wait()
        pltpu.make_async_copy(dst2.at[pl.ds(r0, SB), :], dA, semiA).wait()
        process(sA, dA)

        @pl.when(2 * j + 2 < NSUP)
        def _():
            pltpu.async_copy(src2.at[pl.ds(r0 + 2 * SB, SB), :], sA, semiA)
            pltpu.async_copy(dst2.at[pl.ds(r0 + 2 * SB, SB), :], dA, semiA)

        pltpu.make_async_copy(src2.at[pl.ds(r0 + SB, SB), :], sB,
                              semiB).wait()
        pltpu.make_async_copy(dst2.at[pl.ds(r0 + SB, SB), :], dB,
                              semiB).wait()
        process(sB, dB)

        @pl.when(2 * j + 3 < NSUP)
        def _():
            pltpu.async_copy(src2.at[pl.ds(r0 + 3 * SB, SB), :], sB, semiB)
            pltpu.async_copy(dst2.at[pl.ds(r0 + 3 * SB, SB), :], dB, semiB)

    @pl.when(wid < NEXTRA)
    def _():
        pltpu.sync_copy(src2.at[NFULL * NW + wid], sA.at[0])
        pltpu.sync_copy(dst2.at[NFULL * NW + wid], dA.at[0])
        pltpu.async_copy(table.at[sA.at[0]], rows0, semg0).wait()
        pltpu.sync_copy(rows0, acc.at[dA.at[0]], add=True)

    plsc.subcore_barrier()
    pltpu.sync_copy(acc.at[pl.ds(s * RT, RT), :],
                    out.at[c, pl.ds(s * RT, RT), :])


def _make_agg(D):
    return pl.kernel(
        functools.partial(_agg_body, D),
        out_type=jax.ShapeDtypeStruct((NC, N_PAD, D), _F32),
        mesh=_MESH,
        compiler_params=_SC_PARAMS,
        scratch_types=[
            pltpu.VMEM((SB, CK), jnp.int32),
            pltpu.VMEM((SB, CK), jnp.int32),
            pltpu.VMEM((SB, CK), jnp.int32),
            pltpu.VMEM((SB, CK), jnp.int32),
            pltpu.VMEM((CK, D), _F32),
            pltpu.VMEM((CK, D), _F32),
            pltpu.VMEM_SHARED((N_PAD, D), _F32),
            pltpu.SemaphoreType.DMA,
            pltpu.SemaphoreType.DMA,
            pltpu.SemaphoreType.DMA,
            pltpu.SemaphoreType.DMA,
            pltpu.SemaphoreType.DMA,
            pltpu.SemaphoreType.DMA,
        ],
    )


_agg64 = _make_agg(H)
_agg32 = _make_agg(H2)


def _dense1_body(deg_ref, x_ref, w1_ref, dinv_ref, hh1_ref):
    deg = deg_ref[...]
    dinv = lax.rsqrt(deg[0] + deg[1] + 1.0)
    dinv_ref[...] = dinv[:, None]
    h = jnp.dot(x_ref[...], w1_ref[...], preferred_element_type=_F32,
                precision=_HIGH)
    hh1_ref[...] = h * dinv[:N, None]


def _dense2_body(p_ref, hh1_ref, dinv_ref, b1_ref, w2_ref, hh2_ref):
    dinv = dinv_ref[...][:N]
    p = p_ref[...]
    agg = (p[0, :N] + p[1, :N] + hh1_ref[...]) * dinv
    t1 = jnp.tanh(agg + b1_ref[...])
    hh2_ref[...] = jnp.dot(t1, w2_ref[...], preferred_element_type=_F32,
                           precision=_HIGH) * dinv


def _dense3_body(p_ref, hh2_ref, dinv_ref, b2_ref, wf1_ref, bf1_ref,
                 wf2_ref, bf2_ref, batch_ref, ia_ref, ib_ref,
                 out_ref, xu_ref):
    dinv = dinv_ref[...][:N]
    p = p_ref[...]
    agg = (p[0, :N] + p[1, :N] + hh2_ref[...]) * dinv
    t2 = jnp.tanh(agg + b2_ref[...])
    w = jnp.dot(wf1_ref[...], wf2_ref[...], preferred_element_type=_F32,
                precision=_HIGH)
    cc = jnp.dot(bf1_ref[...], wf2_ref[...], preferred_element_type=_F32,
                 precision=_HIGH) + bf2_ref[...]
    z = jnp.dot(t2, w, preferred_element_type=_F32, precision=_HIGH) + cc

    batch = batch_ref[...]
    sums = jnp.zeros((1, G), _F32)
    cnts = jnp.zeros((1, G), _F32)
    CH = 1250
    for k in range(N // CH):
        bc = batch[k * CH:(k + 1) * CH]
        m = (lax.broadcasted_iota(jnp.int32, (CH, G), 1) == bc).astype(_F32)
        zc = z[k * CH:(k + 1) * CH]
        sums = sums + jnp.sum(m * zc, axis=0, keepdims=True)
        cnts = cnts + jnp.sum(m, axis=0, keepdims=True)
    u = sums / jnp.maximum(cnts, 1.0)
    xu_ref[...] = u.reshape(G, 1)

    gi = lax.broadcasted_iota(jnp.int32, (P, G), 1)
    a = (ib_ref[...] == gi).astype(_F32) - (ia_ref[...] == gi).astype(_F32)
    out_ref[...] = jnp.sum(a * u, axis=1, keepdims=True)


_dense1 = pl.pallas_call(
    _dense1_body,
    out_shape=[jax.ShapeDtypeStruct((N_PAD, 1), _F32),
               jax.ShapeDtypeStruct((N, H), _F32)],
)

_dense2 = pl.pallas_call(
    _dense2_body,
    out_shape=jax.ShapeDtypeStruct((N, H2), _F32),
)

_dense3 = pl.pallas_call(
    _dense3_body,
    out_shape=[jax.ShapeDtypeStruct((P, 1), _F32),
               jax.ShapeDtypeStruct((G, 1), _F32)],
)


def kernel(x, edge_index, batch, idx_a, idx_b, W1, b1, W2, b2,
           Wf1, bf1, Wf2, bf2):
    e2 = edge_index.reshape(2, NCH, CK)
    src2 = e2[0]
    dst2 = e2[1]
    degp = _deg_call(dst2)
    dinv2d, hh1 = _dense1(degp, x, W1)
    p1 = _agg64(src2, dst2, hh1)
    hh2 = _dense2(p1, hh1, dinv2d, b1.reshape(1, H), W2)
    p2 = _agg32(src2, dst2, hh2)
    out2d, xu = _dense3(p2, hh2, dinv2d, b2.reshape(1, H2), Wf1,
                        bf1.reshape(1, H2), Wf2, bf2.reshape(1, 1),
                        batch.reshape(N, 1), idx_a.reshape(P, 1),
                        idx_b.reshape(P, 1))
    return out2d.reshape(P), xu


# speedup vs baseline: 43.6461x; 1.0199x over previous
import functools

import jax
import jax.numpy as jnp
from jax import lax
from jax.experimental import pallas as pl
from jax.experimental.pallas import tpu as pltpu
from jax.experimental.pallas import tpu_sc as plsc

N = 10000
E = 320000
DF = 128
H = 64
H2 = 32
G = 256
P = 512

N_PAD = 10240
NC = 2
NS = 16
NW = NC * NS
CK = 128
NCH = E // CK
NFULL = NCH // NW
NEXTRA = NCH - NFULL * NW
SB = 13
NSUP = NFULL // SB
RT = N_PAD // NS

_MESH = plsc.VectorSubcoreMesh(core_axis_name="c", subcore_axis_name="s")
_F32 = jnp.float32
_HIGH = jax.lax.Precision.HIGHEST


def _zero_f32_vmem(ref, n_rows, n_cols):
    z16 = jnp.zeros((16,), _F32)

    @pl.loop(0, n_rows)
    def _(i):
        for j in range(n_cols // 16):
            ref[i, pl.ds(j * 16, 16)] = z16


def _zero_f32_vmem_1d(ref, n):
    z16 = jnp.zeros((16,), _F32)

    @pl.loop(0, n // 16)
    def _(i):
        ref[pl.ds(i * 16, 16)] = z16


def _deg_body(edges, out, dA, dB, ones_ck, zrow, acc, semiA, semiB, sems):
    c = lax.axis_index("c")
    s = lax.axis_index("s")
    wid = c * NS + s
    base_row = wid * NFULL
    dst2 = edges.at[1]

    pltpu.async_copy(dst2.at[pl.ds(base_row, SB), :], dA, semiA)
    pltpu.async_copy(dst2.at[pl.ds(base_row + SB, SB), :], dB, semiB)

    for j in range(CK // 16):
        ones_ck[pl.ds(j * 16, 16)] = jnp.ones((16,), _F32)
    _zero_f32_vmem_1d(zrow, RT)

    pltpu.sync_copy(zrow, acc.at[pl.ds(s * RT, RT)])
    plsc.subcore_barrier()

    def process(dbuf):
        descs = [pltpu.async_copy(ones_ck, acc.at[dbuf.at[b]], sems,
                                  add=True)
                 for b in range(SB)]
        for d in descs:
            d.wait()

    @pl.loop(0, NSUP // 2)
    def _(j):
        r0 = base_row + 2 * j * SB
        pltpu.make_async_copy(dst2.at[pl.ds(r0, SB), :], dA, semiA).wait()
        process(dA)

        @pl.when(2 * j + 2 < NSUP)
        def _():
            pltpu.async_copy(dst2.at[pl.ds(r0 + 2 * SB, SB), :], dA, semiA)

        pltpu.make_async_copy(dst2.at[pl.ds(r0 + SB, SB), :], dB,
                              semiB).wait()
        process(dB)

        @pl.when(2 * j + 3 < NSUP)
        def _():
            pltpu.async_copy(dst2.at[pl.ds(r0 + 3 * SB, SB), :], dB, semiB)

    @pl.when(wid < NEXTRA)
    def _():
        pltpu.sync_copy(dst2.at[NFULL * NW + wid], dA.at[0])
        pltpu.sync_copy(ones_ck, acc.at[dA.at[0]], add=True)

    plsc.subcore_barrier()
    pltpu.sync_copy(acc.at[pl.ds(s * RT, RT)], out.at[c, pl.ds(s * RT, RT)])


_SC_PARAMS = pltpu.CompilerParams(use_tc_tiling_on_sc=False)

_deg_call = pl.kernel(
    _deg_body,
    out_type=jax.ShapeDtypeStruct((NC, N_PAD), _F32),
    mesh=_MESH,
    compiler_params=_SC_PARAMS,
    scratch_types=[
        pltpu.VMEM((SB, CK), jnp.int32),
        pltpu.VMEM((SB, CK), jnp.int32),
        pltpu.VMEM((CK,), _F32),
        pltpu.VMEM((RT,), _F32),
        pltpu.VMEM_SHARED((N_PAD,), _F32),
        pltpu.SemaphoreType.DMA,
        pltpu.SemaphoreType.DMA,
        pltpu.SemaphoreType.DMA,
    ],
)


def _agg_body(D, edges, table, out, sA, dA, sB, dB, rows0, rows1,
              acc, semiA, semiB, semg0, semg1, sems0, sems1):
    c = lax.axis_index("c")
    s = lax.axis_index("s")
    wid = c * NS + s
    base_row = wid * NFULL
    src2 = edges.at[0]
    dst2 = edges.at[1]

    pltpu.async_copy(src2.at[pl.ds(base_row, SB), :], sA, semiA)
    pltpu.async_copy(dst2.at[pl.ds(base_row, SB), :], dA, semiA)
    pltpu.async_copy(src2.at[pl.ds(base_row + SB, SB), :], sB, semiB)
    pltpu.async_copy(dst2.at[pl.ds(base_row + SB, SB), :], dB, semiB)

    _zero_f32_vmem(rows0, CK, D)

    @pl.loop(0, RT // CK)
    def _(t):
        pltpu.sync_copy(rows0, acc.at[pl.ds(s * RT + t * CK, CK), :])

    plsc.subcore_barrier()

    rbufs = (rows0, rows1)
    semg = (semg0, semg1)
    sems = (sems0, sems1)

    def process(sbuf, dbuf):
        gd = [None, None]
        sd = [None, None]
        gd[0] = pltpu.async_copy(table.at[sbuf.at[0]], rbufs[0], semg[0])
        for b in range(SB):
            p = b % 2
            q = (b + 1) % 2
            if b + 1 < SB:
                if sd[q] is not None:
                    sd[q].wait()
                gd[q] = pltpu.async_copy(table.at[sbuf.at[b + 1]],
                                         rbufs[q], semg[q])
            gd[p].wait()
            sd[p] = pltpu.async_copy(rbufs[p], acc.at[dbuf.at[b]],
                                     sems[p], add=True)
        sd[(SB - 1) % 2].wait()
        sd[SB % 2].wait()

    @pl.loop(0, NSUP // 2)
    def _(j):
        r0 = base_row + 2 * j * SB
        pltpu.make_async_copy(src2.at[pl.ds(r0, SB), :], sA, semiA).wait()
        pltpu.make_async_copy(dst2.at[pl.ds(r0, SB), :], dA, semiA).wait()
        process(sA, dA)

        @pl.when(2 * j + 2 < NSUP)
        def _():
            pltpu.async_copy(src2.at[pl.ds(r0 + 2 * SB, SB), :], sA, semiA)
            pltpu.async_copy(dst2.at[pl.ds(r0 + 2 * SB, SB), :], dA, semiA)

        pltpu.make_async_copy(src2.at[pl.ds(r0 + SB, SB), :], sB,
                              semiB).wait()
        pltpu.make_async_copy(dst2.at[pl.ds(r0 + SB, SB), :], dB,
                              semiB).wait()
        process(sB, dB)

        @pl.when(2 * j + 3 < NSUP)
        def _():
            pltpu.async_copy(src2.at[pl.ds(r0 + 3 * SB, SB), :], sB, semiB)
            pltpu.async_copy(dst2.at[pl.ds(r0 + 3 * SB, SB), :], dB, semiB)

    @pl.when(wid < NEXTRA)
    def _():
        pltpu.sync_copy(src2.at[NFULL * NW + wid], sA.at[0])
        pltpu.sync_copy(dst2.at[NFULL * NW + wid], dA.at[0])
        pltpu.async_copy(table.at[sA.at[0]], rows0, semg0).wait()
        pltpu.sync_copy(rows0, acc.at[dA.at[0]], add=True)

    plsc.subcore_barrier()
    pltpu.sync_copy(acc.at[pl.ds(s * RT, RT), :],
                    out.at[c, pl.ds(s * RT, RT), :])


def _make_agg(D):
    return pl.kernel(
        functools.partial(_agg_body, D),
        out_type=jax.ShapeDtypeStruct((NC, N_PAD, D), _F32),
        mesh=_MESH,
        compiler_params=_SC_PARAMS,
        scratch_types=[
            pltpu.VMEM((SB, CK), jnp.int32),
            pltpu.VMEM((SB, CK), jnp.int32),
            pltpu.VMEM((SB, CK), jnp.int32),
            pltpu.VMEM((SB, CK), jnp.int32),
            pltpu.VMEM((CK, D), _F32),
            pltpu.VMEM((CK, D), _F32),
            pltpu.VMEM_SHARED((N_PAD, D), _F32),
            pltpu.SemaphoreType.DMA,
            pltpu.SemaphoreType.DMA,
            pltpu.SemaphoreType.DMA,
            pltpu.SemaphoreType.DMA,
            pltpu.SemaphoreType.DMA,
            pltpu.SemaphoreType.DMA,
        ],
    )


_agg64 = _make_agg(H)
_agg32 = _make_agg(H2)


def _dense1_body(deg_ref, x_ref, w1_ref, dinv_ref, hh1_ref):
    deg = deg_ref[...]
    dinv = lax.rsqrt(deg[0] + deg[1] + 1.0)
    dinv_ref[...] = dinv[:, None]
    h = jnp.dot(x_ref[...], w1_ref[...], preferred_element_type=_F32,
                precision=_HIGH)
    hh1_ref[...] = h * dinv[:N, None]


def _dense2_body(p_ref, hh1_ref, dinv_ref, b1_ref, w2_ref, hh2_ref):
    dinv = dinv_ref[...][:N]
    p = p_ref[...]
    agg = (p[0, :N] + p[1, :N] + hh1_ref[...]) * dinv
    t1 = jnp.tanh(agg + b1_ref[...])
    hh2_ref[...] = jnp.dot(t1, w2_ref[...], preferred_element_type=_F32,
                           precision=_HIGH) * dinv


def _dense3_body(p_ref, hh2_ref, dinv_ref, b2_ref, wf1_ref, bf1_ref,
                 wf2_ref, bf2_ref, batch_ref, ia_ref, ib_ref,
                 out_ref, xu_ref):
    dinv = dinv_ref[...][:N]
    p = p_ref[...]
    agg = (p[0, :N] + p[1, :N] + hh2_ref[...]) * dinv
    t2 = jnp.tanh(agg + b2_ref[...])
    w = jnp.dot(wf1_ref[...], wf2_ref[...], preferred_element_type=_F32,
                precision=_HIGH)
    cc = jnp.dot(bf1_ref[...], wf2_ref[...], preferred_element_type=_F32,
                 precision=_HIGH) + bf2_ref[...]
    z = jnp.dot(t2, w, preferred_element_type=_F32, precision=_HIGH) + cc

    batch = batch_ref[...]
    zt = z.reshape(1, N)
    sums = jnp.zeros((G, 1), _F32)
    cnts = jnp.zeros((G, 1), _F32)
    CH = 1250
    for k in range(N // CH):
        bc = batch[:, k * CH:(k + 1) * CH]
        m = (lax.broadcasted_iota(jnp.int32, (G, CH), 0) == bc).astype(_F32)
        zc = zt[:, k * CH:(k + 1) * CH]
        sums = sums + jnp.sum(m * zc, axis=1, keepdims=True)
        cnts = cnts + jnp.sum(m, axis=1, keepdims=True)
    u = sums / jnp.maximum(cnts, 1.0)
    xu_ref[...] = u

    gi = lax.broadcasted_iota(jnp.int32, (P, G), 1)
    a = (ib_ref[...] == gi).astype(_F32) - (ia_ref[...] == gi).astype(_F32)
    out_ref[...] = jnp.dot(a, u, preferred_element_type=_F32,
                           precision=_HIGH)


_dense1 = pl.pallas_call(
    _dense1_body,
    out_shape=[jax.ShapeDtypeStruct((N_PAD, 1), _F32),
               jax.ShapeDtypeStruct((N, H), _F32)],
)

_dense2 = pl.pallas_call(
    _dense2_body,
    out_shape=jax.ShapeDtypeStruct((N, H2), _F32),
)

_dense3 = pl.pallas_call(
    _dense3_body,
    out_shape=[jax.ShapeDtypeStruct((P, 1), _F32),
               jax.ShapeDtypeStruct((G, 1), _F32)],
)


def kernel(x, edge_index, batch, idx_a, idx_b, W1, b1, W2, b2,
           Wf1, bf1, Wf2, bf2):
    e2 = edge_index.reshape(2, NCH, CK)
    degp = _deg_call(e2)
    dinv2d, hh1 = _dense1(degp, x, W1)
    p1 = _agg64(e2, hh1)
    hh2 = _dense2(p1, hh1, dinv2d, b1.reshape(1, H), W2)
    p2 = _agg32(e2, hh2)
    out2d, xu = _dense3(p2, hh2, dinv2d, b2.reshape(1, H2), Wf1,
                        bf1.reshape(1, H2), Wf2, bf2.reshape(1, 1),
                        batch.reshape(1, N), idx_a.reshape(P, 1),
                        idx_b.reshape(P, 1))
    return out2d.reshape(P), xu
